# baseline (speedup 1.0000x reference)
import jax
import jax.numpy as jnp
from jax.experimental import pallas as pl
from jax.experimental.pallas import tpu as pltpu

ROWS, COLS = 128, 100000
BLK = 4000
SUB = 200
NBLK = COLS // BLK
NSUB = BLK // SUB
OBLK = 2000
NOBLK = COLS // OBLK


def _rotl(x, d):
    return (x << jnp.uint32(d)) | (x >> jnp.uint32(32 - d))


def _bits_from_x1(x1):
    ks = (jnp.uint32(0), jnp.uint32(1), jnp.uint32(0x1BD11BDB))
    rotations = ((13, 15, 26, 6), (17, 29, 16, 24))
    x0 = x1
    x1 = _rotl(x1, 13)
    x1 = x0 ^ x1
    for r in (15, 26, 6):
        x0 = x0 + x1
        x1 = _rotl(x1, r)
        x1 = x0 ^ x1
    x0 = x0 + ks[1]
    x1 = x1 + ks[2] + jnp.uint32(1)
    for i in range(1, 5):
        for r in rotations[i % 2]:
            x0 = x0 + x1
            x1 = _rotl(x1, r)
            x1 = x0 ^ x1
        x0 = x0 + ks[(i + 1) % 3]
        x1 = x1 + ks[(i + 2) % 3] + jnp.uint32(i + 1)
    return x0 ^ x1


def _w_from_x1(x1):
    bits = _bits_from_x1(x1)
    fbits = (bits >> jnp.uint32(9)) | jnp.uint32(0x3F800000)
    u = jax.lax.bitcast_convert_type(fbits, jnp.float32) - 1.0
    eps = jnp.float32(1e-20)
    return eps - jnp.log(u + eps)


def _kernel_body(inv_t_ref, lt_ref, out_ref, e_buf, acc_m, acc_s, snap):
    i = pl.program_id(0)
    k = i
    row0 = k * BLK

    @pl.when(i < NBLK)
    def _phase0():
        c = inv_t_ref[0]
        r_io = jax.lax.broadcasted_iota(jnp.uint32, (SUB, 128), 0)
        c_io = jax.lax.broadcasted_iota(jnp.uint32, (SUB, 128), 1)
        lane_off = r_io + c_io * jnp.uint32(COLS)
        fresh = k == 0
        m8 = jnp.where(fresh, jnp.full((8, 128), -jnp.inf, jnp.float32),
                       acc_m[...])
        s8 = jnp.where(fresh, jnp.zeros((8, 128), jnp.float32), acc_s[...])
        for j in range(NSUB):
            base = (jnp.asarray(row0, jnp.int32)
                    + jnp.int32(j * SUB + 1)).astype(jnp.uint32)
            w = _w_from_x1(lane_off + base)
            z = (lt_ref[pl.ds(j * SUB, SUB), :] - jnp.log(w)) * c
            z3 = z.reshape(SUB // 8, 8, 128)
            m_new = jnp.maximum(m8, jnp.max(z3, axis=0))
            e3 = jax.lax.exp2(z3 - m_new[None])
            e_buf[pl.ds(row0 + j * SUB, SUB), :] = e3.reshape(SUB, 128)
            s8 = s8 * jax.lax.exp2(m8 - m_new) + jnp.sum(e3, axis=0)
            snap[pl.ds((k * NSUB + j) * 8, 8), :] = m_new
            m8 = m_new
        acc_m[...] = m8
        acc_s[...] = s8

    @pl.when(i >= NBLK)
    def _phase1():
        k2 = i - NBLK
        m8 = acc_m[...]
        s8 = acc_s[...]
        m = jnp.max(m8, axis=0, keepdims=True)
        s = jnp.sum(s8 * jax.lax.exp2(m8 - m), axis=0, keepdims=True)
        inv_s = 1.0 / s
        for j in range(OBLK // SUB):
            cj = k2 * (OBLK // SUB) + j
            f = jax.lax.exp2(snap[pl.ds(cj * 8, 8), :] - m) * inv_s
            e3 = e_buf[pl.ds(cj * SUB, SUB), :].reshape(SUB // 8, 8, 128)
            out_ref[pl.ds(j * SUB, SUB), :] = (e3 * f[None]).reshape(SUB, 128)


@jax.jit
def kernel(logits, temperature):
    inv_t = (jnp.float32(1.4426950408889634)
             / jnp.asarray(temperature, jnp.float32)).reshape(1)
    lt = logits.T
    out_t = pl.pallas_call(
        _kernel_body,
        grid=(NBLK + NOBLK,),
        in_specs=[
            pl.BlockSpec(memory_space=pltpu.SMEM),
            pl.BlockSpec((BLK, ROWS), lambda i: (jnp.where(i < NBLK, i, 0), 0)),
        ],
        out_specs=pl.BlockSpec(
            (OBLK, ROWS), lambda i: (jnp.where(i < NBLK, 0, i - NBLK), 0)),
        out_shape=jax.ShapeDtypeStruct((COLS, ROWS), jnp.float32),
        scratch_shapes=[
            pltpu.VMEM((COLS, ROWS), jnp.float32),
            pltpu.VMEM((8, 128), jnp.float32),
            pltpu.VMEM((8, 128), jnp.float32),
            pltpu.VMEM((NBLK * NSUB * 8, 128), jnp.float32),
        ],
    )(inv_t, lt)
    return out_t.T


# speedup vs baseline: 1.0385x; 1.0385x over previous
import jax
import jax.numpy as jnp
from jax.experimental import pallas as pl
from jax.experimental.pallas import tpu as pltpu

ROWS, COLS = 128, 100000
BLK = 2000
SUB = 200
NBLK = COLS // BLK
NSUB = BLK // SUB
OBLK = 5000
NOBLK = COLS // OBLK


def _rotl(x, d):
    return (x << jnp.uint32(d)) | (x >> jnp.uint32(32 - d))


def _bits_from_x1(x1):
    ks = (jnp.uint32(0), jnp.uint32(1), jnp.uint32(0x1BD11BDB))
    rotations = ((13, 15, 26, 6), (17, 29, 16, 24))
    x0 = x1
    x1 = _rotl(x1, 13)
    x1 = x0 ^ x1
    for r in (15, 26, 6):
        x0 = x0 + x1
        x1 = _rotl(x1, r)
        x1 = x0 ^ x1
    x0 = x0 + ks[1]
    x1 = x1 + ks[2] + jnp.uint32(1)
    for i in range(1, 5):
        for r in rotations[i % 2]:
            x0 = x0 + x1
            x1 = _rotl(x1, r)
            x1 = x0 ^ x1
        x0 = x0 + ks[(i + 1) % 3]
        x1 = x1 + ks[(i + 2) % 3] + jnp.uint32(i + 1)
    return x0 ^ x1


def _w_from_x1(x1):
    bits = _bits_from_x1(x1)
    fbits = (bits >> jnp.uint32(9)) | jnp.uint32(0x3F800000)
    u = jax.lax.bitcast_convert_type(fbits, jnp.float32) - 1.0
    eps = jnp.float32(1e-20)
    return eps - jnp.log(u + eps)


def _kernel_body(inv_t_ref, lt_ref, out_ref, e_buf, acc_m, acc_s, snap):
    i = pl.program_id(0)
    k = i
    row0 = k * BLK

    @pl.when(i < NBLK)
    def _phase0():
        c = inv_t_ref[0]
        r_io = jax.lax.broadcasted_iota(jnp.uint32, (SUB, 128), 0)
        c_io = jax.lax.broadcasted_iota(jnp.uint32, (SUB, 128), 1)
        lane_off = r_io + c_io * jnp.uint32(COLS)
        fresh = k == 0
        m8 = jnp.where(fresh, jnp.full((8, 128), -jnp.inf, jnp.float32),
                       acc_m[...])
        s8 = jnp.where(fresh, jnp.zeros((8, 128), jnp.float32), acc_s[...])
        for j in range(NSUB):
            base = (jnp.asarray(row0, jnp.int32)
                    + jnp.int32(j * SUB + 1)).astype(jnp.uint32)
            w = _w_from_x1(lane_off + base)
            z = (lt_ref[pl.ds(j * SUB, SUB), :] - jnp.log(w)) * c
            z3 = z.reshape(SUB // 8, 8, 128)
            m_new = jnp.maximum(m8, jnp.max(z3, axis=0))
            e3 = jax.lax.exp2(z3 - m_new[None])
            e_buf[pl.ds(row0 + j * SUB, SUB), :] = e3.reshape(SUB, 128)
            s8 = s8 * jax.lax.exp2(m8 - m_new) + jnp.sum(e3, axis=0)
            snap[pl.ds((k * NSUB + j) * 8, 8), :] = m_new
            m8 = m_new
        acc_m[...] = m8
        acc_s[...] = s8

    @pl.when(i >= NBLK)
    def _phase1():
        k2 = i - NBLK
        m8 = acc_m[...]
        s8 = acc_s[...]
        m = jnp.max(m8, axis=0, keepdims=True)
        s = jnp.sum(s8 * jax.lax.exp2(m8 - m), axis=0, keepdims=True)
        inv_s = 1.0 / s
        for j in range(OBLK // SUB):
            cj = k2 * (OBLK // SUB) + j
            f = jax.lax.exp2(snap[pl.ds(cj * 8, 8), :] - m) * inv_s
            e3 = e_buf[pl.ds(cj * SUB, SUB), :].reshape(SUB // 8, 8, 128)
            out_ref[pl.ds(j * SUB, SUB), :] = (e3 * f[None]).reshape(SUB, 128)


@jax.jit
def kernel(logits, temperature):
    inv_t = (jnp.float32(1.4426950408889634)
             / jnp.asarray(temperature, jnp.float32)).reshape(1)
    lt = logits.T
    out_t = pl.pallas_call(
        _kernel_body,
        grid=(NBLK + NOBLK,),
        in_specs=[
            pl.BlockSpec(memory_space=pltpu.SMEM),
            pl.BlockSpec((BLK, ROWS), lambda i: (jnp.where(i < NBLK, i, 0), 0)),
        ],
        out_specs=pl.BlockSpec(
            (OBLK, ROWS), lambda i: (jnp.where(i < NBLK, 0, i - NBLK), 0)),
        out_shape=jax.ShapeDtypeStruct((COLS, ROWS), jnp.float32),
        scratch_shapes=[
            pltpu.VMEM((COLS, ROWS), jnp.float32),
            pltpu.VMEM((8, 128), jnp.float32),
            pltpu.VMEM((8, 128), jnp.float32),
            pltpu.VMEM((NBLK * NSUB * 8, 128), jnp.float32),
        ],
    )(inv_t, lt)
    return out_t.T
